# baseline (speedup 1.0000x reference)
import jax
import jax.numpy as jnp
from jax import lax
from jax.experimental import pallas as pl
from jax.experimental.pallas import tpu as pltpu
from jax.experimental.pallas import tpu_sc as plsc

N_NODES = 10000
N_EDGES = 320000
DIM = 128
NUM_EMB = 4

NC = 2
NS = 16
TILES = NC * NS
TE = N_EDGES // TILES
CHUNK = 80
NCHG = N_EDGES // CHUNK
NCH_BASE = NCHG // TILES
NCH_REM = NCHG % TILES
NBUF = 4
N_PAD = 10112
ROWS_PER_TILE = N_PAD // NS
ZROWS = 640


def _prep_body(x_ref, t_ref, src_ref, attr_ref, xt_ref, idx_ref, z_ref):
    xv = x_ref[...]
    for a in range(NUM_EMB):
        xt_ref[:, a, :] = jnp.maximum(xv + t_ref[a:a + 1, :], 0.0)
    idx_ref[...] = src_ref[...] * NUM_EMB + attr_ref[...]
    z_ref[...] = jnp.zeros(z_ref.shape, z_ref.dtype)


def _prep(x, table, src, attr):
    grid = 10
    bn = N_NODES // grid
    er = N_EDGES // DIM
    bz = ZROWS // grid
    src2 = src.reshape(er, DIM)
    attr2 = attr.reshape(er, DIM)
    return pl.pallas_call(
        _prep_body,
        grid=(grid,),
        in_specs=[
            pl.BlockSpec((bn, DIM), lambda i: (i, 0)),
            pl.BlockSpec((NUM_EMB, DIM), lambda i: (0, 0)),
            pl.BlockSpec((er, DIM), lambda i: (0, 0)),
            pl.BlockSpec((er, DIM), lambda i: (0, 0)),
        ],
        out_specs=[
            pl.BlockSpec((bn, NUM_EMB, DIM), lambda i: (i, 0, 0)),
            pl.BlockSpec((er, DIM), lambda i: (0, 0)),
            pl.BlockSpec((bz, DIM), lambda i: (i, 0)),
        ],
        out_shape=[
            jax.ShapeDtypeStruct((N_NODES, NUM_EMB, DIM), jnp.float32),
            jax.ShapeDtypeStruct((er, DIM), jnp.int32),
            jax.ShapeDtypeStruct((ZROWS, DIM), jnp.float32),
        ],
    )(x, table, src2, attr2)


def _sc_body(xt_hbm, idx_hbm, dst_hbm, zero_hbm, out_hbm,
             idx_v, dst_v, rows_v, sem_i, sem_g, aggr_sh):
    c = lax.axis_index("c")
    s = lax.axis_index("s")
    w = c * NS + s
    pltpu.sync_copy(zero_hbm.at[pl.ds(0, ROWS_PER_TILE)],
                    aggr_sh.at[pl.ds(s * ROWS_PER_TILE, ROWS_PER_TILE)])
    plsc.subcore_barrier()

    nch = NCH_BASE + jnp.where(w < NCH_REM, 1, 0)

    def base_of(k):
        return pl.multiple_of((w + k * TILES) * CHUNK, 8)

    def load_indices(k, b):
        pltpu.async_copy(idx_hbm.at[pl.ds(base_of(k), CHUNK)], idx_v.at[b], sem_i.at[b])
        pltpu.async_copy(dst_hbm.at[pl.ds(base_of(k), CHUNK)], dst_v.at[b], sem_i.at[b])

    def wait_indices(k, b):
        pltpu.make_async_copy(idx_hbm.at[pl.ds(base_of(k), CHUNK)], idx_v.at[b], sem_i.at[b]).wait()
        pltpu.make_async_copy(dst_hbm.at[pl.ds(base_of(k), CHUNK)], dst_v.at[b], sem_i.at[b]).wait()

    def start_gather(b):
        pltpu.async_copy(xt_hbm.at[idx_v.at[b]], rows_v.at[b], sem_g.at[b])

    def wait_gather(b):
        pltpu.make_async_copy(xt_hbm.at[idx_v.at[b]], rows_v.at[b], sem_g.at[b]).wait()

    for j in range(NBUF):
        load_indices(j, j)
    for j in range(NBUF - 1):
        wait_indices(j, j)
        start_gather(j)

    def chunk(k, carry):
        b = lax.rem(k, NBUF)
        wait_gather(b)

        @pl.when(k + NBUF - 1 < nch)
        def _():
            nb = lax.rem(k + NBUF - 1, NBUF)
            wait_indices(k + NBUF - 1, nb)
            start_gather(nb)

        pltpu.sync_copy(rows_v.at[b], aggr_sh.at[dst_v.at[b]], add=True)

        @pl.when(k + NBUF < nch)
        def _():
            load_indices(k + NBUF, b)

        return carry

    lax.fori_loop(0, nch, chunk, 0)
    plsc.subcore_barrier()
    pltpu.sync_copy(aggr_sh.at[pl.ds(s * ROWS_PER_TILE, ROWS_PER_TILE)],
                    out_hbm.at[c, pl.ds(s * ROWS_PER_TILE, ROWS_PER_TILE)])


def _scatter(xt, idx, dst, zeros):
    fn = pl.kernel(
        _sc_body,
        out_type=jax.ShapeDtypeStruct((NC, N_PAD, DIM), jnp.float32),
        mesh=plsc.VectorSubcoreMesh(core_axis_name="c", subcore_axis_name="s"),
        scratch_types=[
            pltpu.VMEM((NBUF, CHUNK), jnp.int32),
            pltpu.VMEM((NBUF, CHUNK), jnp.int32),
            pltpu.VMEM((NBUF, CHUNK, DIM), jnp.float32),
            pltpu.SemaphoreType.DMA((NBUF,)),
            pltpu.SemaphoreType.DMA((NBUF,)),
            pltpu.VMEM_SHARED((N_PAD, DIM), jnp.float32),
        ],
    )
    return fn(xt, idx, dst, zeros)


def _mlp_body(x_ref, p_ref, w1_ref, b1_ref, g_ref, be_ref, w2_ref, b2_ref,
              eps_ref, o_ref):
    h = x_ref[...] * (1.0 + eps_ref[0]) + p_ref[0, :N_NODES, :] + p_ref[1, :N_NODES, :]
    h1 = jnp.dot(h, w1_ref[...], preferred_element_type=jnp.float32) + b1_ref[...]
    mu = jnp.mean(h1, axis=0, keepdims=True)
    var = jnp.mean(jnp.square(h1 - mu), axis=0, keepdims=True)
    hn = (h1 - mu) / jnp.sqrt(var + 1e-5) * g_ref[...] + be_ref[...]
    h2 = jnp.maximum(hn, 0.0)
    o_ref[...] = jnp.dot(h2, w2_ref[...], preferred_element_type=jnp.float32) + b2_ref[...]


def _mlp(x, partials, W1, b1, gamma, beta, W2, b2, eps):
    return pl.pallas_call(
        _mlp_body,
        in_specs=[
            pl.BlockSpec(memory_space=pltpu.VMEM),
            pl.BlockSpec(memory_space=pltpu.VMEM),
            pl.BlockSpec(memory_space=pltpu.VMEM),
            pl.BlockSpec(memory_space=pltpu.VMEM),
            pl.BlockSpec(memory_space=pltpu.VMEM),
            pl.BlockSpec(memory_space=pltpu.VMEM),
            pl.BlockSpec(memory_space=pltpu.VMEM),
            pl.BlockSpec(memory_space=pltpu.VMEM),
            pl.BlockSpec(memory_space=pltpu.SMEM),
        ],
        out_shape=jax.ShapeDtypeStruct((N_NODES, DIM), jnp.float32),
    )(x, partials, W1, b1.reshape(1, DIM), gamma.reshape(1, DIM),
      beta.reshape(1, DIM), W2, b2.reshape(1, DIM), eps.reshape(1))


def kernel(x, edge_index, edge_attr, W1, b1, gamma, beta, W2, b2, edge_table, eps):
    src = edge_index[0].astype(jnp.int32)
    dst = edge_index[1].astype(jnp.int32)
    attr = edge_attr.astype(jnp.int32)
    xt3, idx2, zeros = _prep(x, edge_table, src, attr)
    xt = xt3.reshape(N_NODES * NUM_EMB, DIM)
    idx = idx2.reshape(N_EDGES)
    partials = _scatter(xt, idx, dst, zeros)
    return _mlp(x, partials, W1, b1, gamma, beta, W2, b2, eps)


# speedup vs baseline: 1.2547x; 1.2547x over previous
import jax
import jax.numpy as jnp
from jax import lax
from jax.experimental import pallas as pl
from jax.experimental.pallas import tpu as pltpu
from jax.experimental.pallas import tpu_sc as plsc

N_NODES = 10000
N_EDGES = 320000
DIM = 128
NUM_EMB = 4

NC = 2
NS = 16
TILES = NC * NS
TE = N_EDGES // TILES
CHUNK = 128
NCHG = N_EDGES // CHUNK
NCH_BASE = NCHG // TILES
NCH_REM = NCHG % TILES
NBUF = 3
DBUF = 2 * NBUF
ROWS_A = 632
ROWS_B = N_NODES - 15 * ROWS_A
ZROWS = 640


def _prep_body(x_ref, t_ref, src_ref, attr_ref, xt_ref, idx_ref, z_ref):
    xv = x_ref[...]
    for a in range(NUM_EMB):
        xt_ref[:, a, :] = jnp.maximum(xv + t_ref[a:a + 1, :], 0.0)
    idx_ref[...] = src_ref[...] * NUM_EMB + attr_ref[...]
    z_ref[...] = jnp.zeros(z_ref.shape, z_ref.dtype)


def _prep(x, table, src, attr):
    grid = 10
    bn = N_NODES // grid
    er = N_EDGES // DIM
    bz = ZROWS // grid
    src2 = src.reshape(er, DIM)
    attr2 = attr.reshape(er, DIM)
    return pl.pallas_call(
        _prep_body,
        grid=(grid,),
        in_specs=[
            pl.BlockSpec((bn, DIM), lambda i: (i, 0)),
            pl.BlockSpec((NUM_EMB, DIM), lambda i: (0, 0)),
            pl.BlockSpec((er, DIM), lambda i: (0, 0)),
            pl.BlockSpec((er, DIM), lambda i: (0, 0)),
        ],
        out_specs=[
            pl.BlockSpec((bn, NUM_EMB, DIM), lambda i: (i, 0, 0)),
            pl.BlockSpec((er, DIM), lambda i: (0, 0)),
            pl.BlockSpec((bz, DIM), lambda i: (i, 0)),
        ],
        out_shape=[
            jax.ShapeDtypeStruct((N_NODES, NUM_EMB, DIM), jnp.float32),
            jax.ShapeDtypeStruct((er, DIM), jnp.int32),
            jax.ShapeDtypeStruct((ZROWS, DIM), jnp.float32),
        ],
    )(x, table, src2, attr2)


def _sc_body(xt_hbm, idx_hbm, dst_hbm, zero_hbm, out_hbm,
             idx_v, dst_v, rows_v, sem_i, sem_g, sem_s, aggr_sh):
    c = lax.axis_index("c")
    s = lax.axis_index("s")
    w = c * NS + s

    @pl.when(s < NS - 1)
    def _():
        pltpu.sync_copy(zero_hbm.at[pl.ds(0, ROWS_A)],
                        aggr_sh.at[pl.ds(s * ROWS_A, ROWS_A)])

    @pl.when(s == NS - 1)
    def _():
        pltpu.sync_copy(zero_hbm.at[pl.ds(0, ROWS_B)],
                        aggr_sh.at[pl.ds((NS - 1) * ROWS_A, ROWS_B)])

    plsc.subcore_barrier()

    nch = NCH_BASE + jnp.where(w < NCH_REM, 1, 0)

    def base_of(k):
        return pl.multiple_of((w + k * TILES) * CHUNK, 8)

    def load_indices(k):
        bi = lax.rem(k, NBUF)
        bd = lax.rem(k, DBUF)
        pltpu.async_copy(idx_hbm.at[pl.ds(base_of(k), CHUNK)], idx_v.at[bi], sem_i.at[bi])
        pltpu.async_copy(dst_hbm.at[pl.ds(base_of(k), CHUNK)], dst_v.at[bd], sem_i.at[bi])

    def wait_indices(k):
        bi = lax.rem(k, NBUF)
        bd = lax.rem(k, DBUF)
        pltpu.make_async_copy(idx_hbm.at[pl.ds(base_of(k), CHUNK)], idx_v.at[bi], sem_i.at[bi]).wait()
        pltpu.make_async_copy(dst_hbm.at[pl.ds(base_of(k), CHUNK)], dst_v.at[bd], sem_i.at[bi]).wait()

    def start_gather(k):
        b = lax.rem(k, NBUF)
        pltpu.async_copy(xt_hbm.at[idx_v.at[b]], rows_v.at[b], sem_g.at[b])

    def wait_gather(k):
        b = lax.rem(k, NBUF)
        pltpu.make_async_copy(xt_hbm.at[idx_v.at[b]], rows_v.at[b], sem_g.at[b]).wait()

    def start_scatter(k):
        b = lax.rem(k, NBUF)
        bd = lax.rem(k, DBUF)
        pltpu.async_copy(rows_v.at[b], aggr_sh.at[dst_v.at[bd]], sem_s.at[b], add=True)

    def wait_scatter(k):
        b = lax.rem(k, NBUF)
        bd = lax.rem(k, DBUF)
        pltpu.make_async_copy(rows_v.at[b], aggr_sh.at[dst_v.at[bd]], sem_s.at[b]).wait()

    load_indices(0)
    load_indices(1)
    load_indices(2)
    wait_indices(0)
    start_gather(0)
    wait_indices(1)
    start_gather(1)

    def chunk(k, carry):
        wait_gather(k)

        @pl.when(k >= 1)
        def _():
            wait_scatter(k - 1)

        @pl.when(k + 2 < nch)
        def _():
            wait_indices(k + 2)
            start_gather(k + 2)

        start_scatter(k)

        @pl.when(k + 3 < nch)
        def _():
            load_indices(k + 3)

        return carry

    lax.fori_loop(0, nch, chunk, 0)
    wait_scatter(nch - 1)
    plsc.subcore_barrier()

    @pl.when(s < NS - 1)
    def _():
        pltpu.sync_copy(aggr_sh.at[pl.ds(s * ROWS_A, ROWS_A)],
                        out_hbm.at[c, pl.ds(s * ROWS_A, ROWS_A)])

    @pl.when(s == NS - 1)
    def _():
        pltpu.sync_copy(aggr_sh.at[pl.ds((NS - 1) * ROWS_A, ROWS_B)],
                        out_hbm.at[c, pl.ds((NS - 1) * ROWS_A, ROWS_B)])


def _scatter(xt, idx, dst, zeros):
    fn = pl.kernel(
        _sc_body,
        out_type=jax.ShapeDtypeStruct((NC, N_NODES, DIM), jnp.float32),
        mesh=plsc.VectorSubcoreMesh(core_axis_name="c", subcore_axis_name="s"),
        scratch_types=[
            pltpu.VMEM((NBUF, CHUNK), jnp.int32),
            pltpu.VMEM((DBUF, CHUNK), jnp.int32),
            pltpu.VMEM((NBUF, CHUNK, DIM), jnp.float32),
            pltpu.SemaphoreType.DMA((NBUF,)),
            pltpu.SemaphoreType.DMA((NBUF,)),
            pltpu.SemaphoreType.DMA((NBUF,)),
            pltpu.VMEM_SHARED((N_NODES, DIM), jnp.float32),
        ],
    )
    return fn(xt, idx, dst, zeros)


def _mlp_body(x_ref, p_ref, w1_ref, b1_ref, g_ref, be_ref, w2_ref, b2_ref,
              eps_ref, o_ref):
    h = x_ref[...] * (1.0 + eps_ref[0]) + p_ref[0] + p_ref[1]
    h1 = jnp.dot(h, w1_ref[...], preferred_element_type=jnp.float32) + b1_ref[...]
    mu = jnp.mean(h1, axis=0, keepdims=True)
    var = jnp.mean(jnp.square(h1 - mu), axis=0, keepdims=True)
    hn = (h1 - mu) / jnp.sqrt(var + 1e-5) * g_ref[...] + be_ref[...]
    h2 = jnp.maximum(hn, 0.0)
    o_ref[...] = jnp.dot(h2, w2_ref[...], preferred_element_type=jnp.float32) + b2_ref[...]


def _mlp(x, partials, W1, b1, gamma, beta, W2, b2, eps):
    return pl.pallas_call(
        _mlp_body,
        in_specs=[
            pl.BlockSpec(memory_space=pltpu.VMEM),
            pl.BlockSpec(memory_space=pltpu.VMEM),
            pl.BlockSpec(memory_space=pltpu.VMEM),
            pl.BlockSpec(memory_space=pltpu.VMEM),
            pl.BlockSpec(memory_space=pltpu.VMEM),
            pl.BlockSpec(memory_space=pltpu.VMEM),
            pl.BlockSpec(memory_space=pltpu.VMEM),
            pl.BlockSpec(memory_space=pltpu.VMEM),
            pl.BlockSpec(memory_space=pltpu.SMEM),
        ],
        out_shape=jax.ShapeDtypeStruct((N_NODES, DIM), jnp.float32),
    )(x, partials, W1, b1.reshape(1, DIM), gamma.reshape(1, DIM),
      beta.reshape(1, DIM), W2, b2.reshape(1, DIM), eps.reshape(1))


def kernel(x, edge_index, edge_attr, W1, b1, gamma, beta, W2, b2, edge_table, eps):
    src = edge_index[0].astype(jnp.int32)
    dst = edge_index[1].astype(jnp.int32)
    attr = edge_attr.astype(jnp.int32)
    xt3, idx2, zeros = _prep(x, edge_table, src, attr)
    xt = xt3.reshape(N_NODES * NUM_EMB, DIM)
    idx = idx2.reshape(N_EDGES)
    partials = _scatter(xt, idx, dst, zeros)
    return _mlp(x, partials, W1, b1, gamma, beta, W2, b2, eps)


# speedup vs baseline: 1.2804x; 1.0204x over previous
import jax
import jax.numpy as jnp
from jax import lax
from jax.experimental import pallas as pl
from jax.experimental.pallas import tpu as pltpu
from jax.experimental.pallas import tpu_sc as plsc

N_NODES = 10000
N_EDGES = 320000
DIM = 128
NUM_EMB = 4

NC = 2
NS = 16
TILES = NC * NS
TE = N_EDGES // TILES
CHUNK = 128
NCHG = N_EDGES // CHUNK
NCH_BASE = NCHG // TILES
NCH_REM = NCHG % TILES
NBUF = 3
DBUF = 2 * NBUF
ROWS_A = 632
ROWS_B = N_NODES - 15 * ROWS_A
ZROWS = 640


def _prep_body(x_ref, t_ref, src_ref, attr_ref, xt_ref, idx_ref, z_ref):
    xv = x_ref[...]
    for a in range(NUM_EMB):
        xt_ref[:, a, :] = jnp.maximum(xv + t_ref[a:a + 1, :], 0.0)
    idx_ref[...] = src_ref[...] * NUM_EMB + attr_ref[...]
    z_ref[...] = jnp.zeros(z_ref.shape, z_ref.dtype)


def _prep(x, table, src, attr):
    grid = 10
    bn = N_NODES // grid
    er = N_EDGES // DIM
    bz = ZROWS // grid
    src2 = src.reshape(er, DIM)
    attr2 = attr.reshape(er, DIM)
    return pl.pallas_call(
        _prep_body,
        grid=(grid,),
        in_specs=[
            pl.BlockSpec((bn, DIM), lambda i: (i, 0)),
            pl.BlockSpec((NUM_EMB, DIM), lambda i: (0, 0)),
            pl.BlockSpec((er, DIM), lambda i: (0, 0)),
            pl.BlockSpec((er, DIM), lambda i: (0, 0)),
        ],
        out_specs=[
            pl.BlockSpec((bn, NUM_EMB, DIM), lambda i: (i, 0, 0)),
            pl.BlockSpec((er, DIM), lambda i: (0, 0)),
            pl.BlockSpec((bz, DIM), lambda i: (i, 0)),
        ],
        out_shape=[
            jax.ShapeDtypeStruct((N_NODES, NUM_EMB, DIM), jnp.float32),
            jax.ShapeDtypeStruct((er, DIM), jnp.int32),
            jax.ShapeDtypeStruct((ZROWS, DIM), jnp.float32),
        ],
    )(x, table, src2, attr2)


def _sc_body(xt_hbm, idx_hbm, dst_hbm, zero_hbm, out_hbm,
             idx_v, dst_v, rows_v, sem_i, sem_g, sem_s, aggr_sh):
    c = lax.axis_index("c")
    s = lax.axis_index("s")
    w = c * NS + s

    nch = NCH_BASE + jnp.where(w < NCH_REM, 1, 0)

    def base_of(k):
        return pl.multiple_of((w + k * TILES) * CHUNK, 8)

    def load_indices(k):
        bi = lax.rem(k, NBUF)
        bd = lax.rem(k, DBUF)
        pltpu.async_copy(idx_hbm.at[pl.ds(base_of(k), CHUNK)], idx_v.at[bi], sem_i.at[bi])
        pltpu.async_copy(dst_hbm.at[pl.ds(base_of(k), CHUNK)], dst_v.at[bd], sem_i.at[bi])

    def wait_indices(k):
        bi = lax.rem(k, NBUF)
        bd = lax.rem(k, DBUF)
        pltpu.make_async_copy(idx_hbm.at[pl.ds(base_of(k), CHUNK)], idx_v.at[bi], sem_i.at[bi]).wait()
        pltpu.make_async_copy(dst_hbm.at[pl.ds(base_of(k), CHUNK)], dst_v.at[bd], sem_i.at[bi]).wait()

    def start_gather(k):
        b = lax.rem(k, NBUF)
        pltpu.async_copy(xt_hbm.at[idx_v.at[b]], rows_v.at[b], sem_g.at[b])

    def wait_gather(k):
        b = lax.rem(k, NBUF)
        pltpu.make_async_copy(xt_hbm.at[idx_v.at[b]], rows_v.at[b], sem_g.at[b]).wait()

    def start_scatter(k):
        b = lax.rem(k, NBUF)
        bd = lax.rem(k, DBUF)
        pltpu.async_copy(rows_v.at[b], aggr_sh.at[dst_v.at[bd]], sem_s.at[b], add=True)

    def wait_scatter(k):
        b = lax.rem(k, NBUF)
        bd = lax.rem(k, DBUF)
        pltpu.make_async_copy(rows_v.at[b], aggr_sh.at[dst_v.at[bd]], sem_s.at[b]).wait()

    load_indices(0)
    load_indices(1)
    load_indices(2)
    wait_indices(0)
    start_gather(0)
    wait_indices(1)
    start_gather(1)

    @pl.when(s < NS - 1)
    def _():
        pltpu.sync_copy(zero_hbm.at[pl.ds(0, ROWS_A)],
                        aggr_sh.at[pl.ds(s * ROWS_A, ROWS_A)])

    @pl.when(s == NS - 1)
    def _():
        pltpu.sync_copy(zero_hbm.at[pl.ds(0, ROWS_B)],
                        aggr_sh.at[pl.ds((NS - 1) * ROWS_A, ROWS_B)])

    plsc.subcore_barrier()

    def chunk(k, carry):
        wait_gather(k)

        @pl.when(k >= 1)
        def _():
            wait_scatter(k - 1)

        @pl.when(k + 2 < nch)
        def _():
            wait_indices(k + 2)
            start_gather(k + 2)

        start_scatter(k)

        @pl.when(k + 3 < nch)
        def _():
            load_indices(k + 3)

        return carry

    lax.fori_loop(0, nch, chunk, 0)
    wait_scatter(nch - 1)
    plsc.subcore_barrier()

    @pl.when(s < NS - 1)
    def _():
        pltpu.sync_copy(aggr_sh.at[pl.ds(s * ROWS_A, ROWS_A)],
                        out_hbm.at[c, pl.ds(s * ROWS_A, ROWS_A)])

    @pl.when(s == NS - 1)
    def _():
        pltpu.sync_copy(aggr_sh.at[pl.ds((NS - 1) * ROWS_A, ROWS_B)],
                        out_hbm.at[c, pl.ds((NS - 1) * ROWS_A, ROWS_B)])


def _scatter(xt, idx, dst, zeros):
    fn = pl.kernel(
        _sc_body,
        out_type=jax.ShapeDtypeStruct((NC, N_NODES, DIM), jnp.float32),
        mesh=plsc.VectorSubcoreMesh(core_axis_name="c", subcore_axis_name="s"),
        scratch_types=[
            pltpu.VMEM((NBUF, CHUNK), jnp.int32),
            pltpu.VMEM((DBUF, CHUNK), jnp.int32),
            pltpu.VMEM((NBUF, CHUNK, DIM), jnp.float32),
            pltpu.SemaphoreType.DMA((NBUF,)),
            pltpu.SemaphoreType.DMA((NBUF,)),
            pltpu.SemaphoreType.DMA((NBUF,)),
            pltpu.VMEM_SHARED((N_NODES, DIM), jnp.float32),
        ],
    )
    return fn(xt, idx, dst, zeros)


def _mlp_body(x_ref, p_ref, w1_ref, b1_ref, g_ref, be_ref, w2_ref, b2_ref,
              eps_ref, o_ref):
    h = x_ref[...] * (1.0 + eps_ref[0]) + p_ref[0] + p_ref[1]
    h1 = jnp.dot(h, w1_ref[...], preferred_element_type=jnp.float32) + b1_ref[...]
    mu = jnp.mean(h1, axis=0, keepdims=True)
    var = jnp.mean(jnp.square(h1 - mu), axis=0, keepdims=True)
    hn = (h1 - mu) / jnp.sqrt(var + 1e-5) * g_ref[...] + be_ref[...]
    h2 = jnp.maximum(hn, 0.0)
    o_ref[...] = jnp.dot(h2, w2_ref[...], preferred_element_type=jnp.float32) + b2_ref[...]


def _mlp(x, partials, W1, b1, gamma, beta, W2, b2, eps):
    return pl.pallas_call(
        _mlp_body,
        in_specs=[
            pl.BlockSpec(memory_space=pltpu.VMEM),
            pl.BlockSpec(memory_space=pltpu.VMEM),
            pl.BlockSpec(memory_space=pltpu.VMEM),
            pl.BlockSpec(memory_space=pltpu.VMEM),
            pl.BlockSpec(memory_space=pltpu.VMEM),
            pl.BlockSpec(memory_space=pltpu.VMEM),
            pl.BlockSpec(memory_space=pltpu.VMEM),
            pl.BlockSpec(memory_space=pltpu.VMEM),
            pl.BlockSpec(memory_space=pltpu.SMEM),
        ],
        out_shape=jax.ShapeDtypeStruct((N_NODES, DIM), jnp.float32),
    )(x, partials, W1, b1.reshape(1, DIM), gamma.reshape(1, DIM),
      beta.reshape(1, DIM), W2, b2.reshape(1, DIM), eps.reshape(1))


def kernel(x, edge_index, edge_attr, W1, b1, gamma, beta, W2, b2, edge_table, eps):
    src = edge_index[0].astype(jnp.int32)
    dst = edge_index[1].astype(jnp.int32)
    attr = edge_attr.astype(jnp.int32)
    xt3, idx2, zeros = _prep(x, edge_table, src, attr)
    xt = xt3.reshape(N_NODES * NUM_EMB, DIM)
    idx = idx2.reshape(N_EDGES)
    partials = _scatter(xt, idx, dst, zeros)
    return _mlp(x, partials, W1, b1, gamma, beta, W2, b2, eps)
